# baseline (speedup 1.0000x reference)
import functools

import jax
import jax.numpy as jnp
from jax import lax
from jax.experimental import pallas as pl
from jax.experimental.pallas import tpu as pltpu
from jax.experimental.pallas import tpu_sc as plsc

_B = 512
_N = 16
_L = 16
_NC = 2
_NS = 16
_S = _N * _N
_CH = _B // _L


def _build_sc_call():
    mesh = plsc.VectorSubcoreMesh(core_axis_name="c", subcore_axis_name="s")

    @functools.partial(
        pl.kernel,
        mesh=mesh,
        out_type=jax.ShapeDtypeStruct((_S, _CH, _L), jnp.float32),
        scratch_types=[
            pltpu.VMEM((_S, _L), jnp.float32),
            pltpu.VMEM((_S, _L), jnp.float32),
        ],
    )
    def sc_stick_breaking(x_hbm, out_hbm, x_v, out_v):
        wid = lax.axis_index("s") * _NC + lax.axis_index("c")
        pltpu.sync_copy(x_hbm.at[:, wid, :], x_v)

        zero = jnp.zeros((_L,), jnp.float32)
        one = jnp.ones((_L,), jnp.float32)

        def row_body(m, col_sums):
            suffix = [zero] * _N
            acc = zero
            for n in range(_N - 1, 0, -1):
                acc = acc + col_sums[n]
                suffix[n - 1] = acc
            sum_row = zero
            new_cols = list(col_sums)
            for n in range(_N):
                xv = x_v[m * _N + n, :]
                bv = one / (one + jnp.exp(-xv))
                cn = jnp.full((_L,), float(2 - _N + n), jnp.float32)
                lower = jnp.maximum(zero, cn - sum_row + suffix[n])
                upper = jnp.minimum(one - sum_row, one - new_cols[n])
                p = lower + bv * (upper - lower)
                out_v[m * _N + n, :] = p
                sum_row = sum_row + p
                new_cols[n] = new_cols[n] + p
            return tuple(new_cols)

        lax.fori_loop(0, _N, row_body, tuple([zero] * _N))
        pltpu.sync_copy(out_v, out_hbm.at[:, wid, :])

    return sc_stick_breaking


_SC_CALL = _build_sc_call()


def kernel(x):
    xt = jnp.transpose(x.reshape(_CH, _L, _S), (2, 0, 1))
    out = _SC_CALL(xt)
    out = jnp.transpose(out, (1, 2, 0))
    return out.reshape(_B, _N, _N)
